# baseline (speedup 1.0000x reference)
import jax
import jax.numpy as jnp
from jax import lax
from jax.experimental import pallas as pl
from jax.experimental.pallas import tpu as pltpu
from jax.experimental.pallas import tpu_sc as plsc

_VOCAB = 100000
_EMB = 64
_CTX2 = 20
_HID = 512

_BLK = 4000
_VTC = 68000
_VSC = _VOCAB - _VTC
_NBTC = _VTC // _BLK
_NBSC = _VSC // _BLK
_NB = _VOCAB // _BLK

_NTEC = 32
_NR = _VSC // _NTEC
_CH = 40
_NCH = _NR // _CH
assert _NR % _CH == 0 and _CH % 8 == 0


def _gather_body(idx_hbm, table_hbm, out_hbm, idx_v, rows_v, sem):
    wid = lax.axis_index("s") * 2 + lax.axis_index("c")

    @pl.when(wid == 0)
    def _():
        pltpu.sync_copy(idx_hbm, idx_v)
        v_lo = idx_v[pl.ds(0, 16)]
        v_hi = idx_v[pl.ds(4, 16)]
        copies = []
        for j in range(_CTX2):
            r = v_lo[j] if j < 16 else v_hi[j - 4]
            copies.append(pltpu.async_copy(
                table_hbm.at[pl.ds(r, 1)], rows_v.at[pl.ds(j, 1)], sem))
        for c in copies:
            c.wait()
        pltpu.sync_copy(rows_v, out_hbm)


def _sc_gather(idx, emb):
    mesh = plsc.VectorSubcoreMesh(core_axis_name="c", subcore_axis_name="s")
    return pl.kernel(
        _gather_body,
        out_type=jax.ShapeDtypeStruct((_CTX2, _EMB), jnp.float32),
        mesh=mesh,
        scratch_types=[
            pltpu.VMEM((_CTX2,), jnp.int32),
            pltpu.VMEM((_CTX2, _EMB), jnp.float32),
            pltpu.SemaphoreType.DMA,
        ],
    )(idx, emb)


def _h_body(x_ref, w1_ref, b1_ref, h_ref):
    v = lax.dot_general(x_ref[...], w1_ref[...], (((1,), (1,)), ((), ())),
                        preferred_element_type=jnp.float32)
    h_ref[...] = jnp.maximum(v + b1_ref[...], 0.0)


def _tc_h(x, W1, b1):
    return pl.pallas_call(
        _h_body,
        out_shape=jax.ShapeDtypeStruct((1, _HID), jnp.float32),
    )(x, W1, b1)


def _sc_logits_body(h_hbm, w2_hbm, out_hbm, h_v, buf, lg, tp, sem0, sem1):
    wid = lax.axis_index("s") * 2 + lax.axis_index("c")
    base = _VTC + wid * _NR
    obase = wid * _NR
    pltpu.sync_copy(h_hbm, h_v)
    hv = [h_v[pl.ds(c * 16, 16)] for c in range(_HID // 16)]
    iota = lax.iota(jnp.int32, 16)

    group_starts = []
    g = 0
    while g + 16 <= _CH:
        group_starts.append(g)
        g += 16
    if _CH % 16:
        group_starts.append(_CH - 16)

    def _start(ci, p):
        sem = [sem0, sem1][p]
        pltpu.async_copy(
            w2_hbm.at[pl.ds(base + ci * _CH, _CH)], buf.at[p], sem)

    def _wait(p):
        sem = [sem0, sem1][p]
        pltpu.make_async_copy(
            w2_hbm.at[pl.ds(base, _CH)], buf.at[p], sem).wait()

    _start(0, 0)
    _start(1, 1)

    def _chunk(ci, p):
        _wait(p)
        for gs in group_starts:
            accs = []
            for r in range(16):
                acc = jnp.zeros((16,), jnp.float32)
                for c in range(_HID // 16):
                    w = buf[p, gs + r, pl.ds(c * 16, 16)]
                    acc = acc + w * hv[c]
                accs.append(acc)
            for r in range(16):
                plsc.store_scatter(
                    tp, [iota, jnp.full((16,), r, jnp.int32)], accs[r])
            vsum = tp[0]
            for j in range(1, 16):
                vsum = vsum + tp[j]
            lg[pl.ds(gs, 16)] = vsum

        @pl.when(ci + 2 < _NCH)
        def _():
            _start(ci + 2, p)

        pltpu.sync_copy(
            lg, out_hbm.at[pl.ds(obase + ci * _CH, _CH)])

    def _loop_body(ci, carry):
        @pl.when(ci % 2 == 0)
        def _():
            _chunk(ci, 0)

        @pl.when(ci % 2 == 1)
        def _():
            _chunk(ci, 1)

        return carry

    lax.fori_loop(0, _NCH, _loop_body, 0)


def _sc_logits(h, W2):
    mesh = plsc.VectorSubcoreMesh(core_axis_name="c", subcore_axis_name="s")
    return pl.kernel(
        _sc_logits_body,
        out_type=jax.ShapeDtypeStruct((_VSC,), jnp.float32),
        mesh=mesh,
        compiler_params=pltpu.CompilerParams(needs_layout_passes=False),
        scratch_types=[
            pltpu.VMEM((_HID,), jnp.float32),
            pltpu.VMEM((2, _CH, _HID), jnp.float32),
            pltpu.VMEM((_CH,), jnp.float32),
            pltpu.VMEM((16, 16), jnp.float32),
            pltpu.SemaphoreType.DMA,
            pltpu.SemaphoreType.DMA,
        ],
    )(h, W2)


def _mlp_body(h_ref, w2_ref, b2_ref, out_ref, st_ref, ms_ref):
    i = pl.program_id(0)

    @pl.when(i == 0)
    def _():
        ms_ref[0] = jnp.float32(-jnp.inf)
        ms_ref[1] = jnp.float32(0.0)

    logits = lax.dot_general(
        h_ref[...], w2_ref[...].astype(jnp.bfloat16), (((1,), (1,)), ((), ())),
        preferred_element_type=jnp.float32)
    blk = logits + b2_ref[0]
    out_ref[pl.ds(i, 1)] = blk.reshape(1, 1, _BLK)
    m_old = ms_ref[0]
    m_new = jnp.maximum(m_old, jnp.max(blk))
    blk_sum = jnp.sum(jnp.exp(blk - m_new))
    scale = jnp.exp(jnp.broadcast_to(m_old - m_new, (1, 128)))[0, 0]
    ms_ref[0] = m_new
    ms_ref[1] = ms_ref[1] * scale + blk_sum

    @pl.when(i == _NBTC - 1)
    def _():
        ii = lax.broadcasted_iota(jnp.int32, (1, 128), 1)
        st_ref[...] = jnp.where(ii == 0, ms_ref[0], ms_ref[1])


def _tc_logits(h_bf, W2, b2r):
    return pl.pallas_call(
        _mlp_body,
        grid=(_NBTC,),
        in_specs=[
            pl.BlockSpec((1, _HID), lambda i: (0, 0)),
            pl.BlockSpec((_BLK, _HID), lambda i: (i, 0)),
            pl.BlockSpec((1, 1, _BLK), lambda i: (i, 0, 0)),
        ],
        out_specs=[
            pl.BlockSpec((_NBTC, 1, _BLK), lambda i: (0, 0, 0)),
            pl.BlockSpec((1, 128), lambda i: (0, 0)),
        ],
        out_shape=[
            jax.ShapeDtypeStruct((_NBTC, 1, _BLK), jnp.float32),
            jax.ShapeDtypeStruct((1, 128), jnp.float32),
        ],
        scratch_shapes=[pltpu.SMEM((2,), jnp.float32)],
    )(h_bf, W2, b2r)


def _combine_body(ltc_ref, lsc_ref, b2_ref, st_ref, out_ref):
    m1 = st_ref[0, 0]
    s1 = st_ref[0, 1]

    def _amax(j, m2):
        c = lsc_ref[pl.ds(j, 1)] + b2_ref[pl.ds(_NBTC + j, 1)]
        out_ref[pl.ds(_NBTC + j, 1)] = c
        return jnp.maximum(m2, jnp.max(c))

    m2 = lax.fori_loop(0, _NBSC, _amax, jnp.float32(-jnp.inf))
    m = jnp.maximum(m1, m2)

    def _asum(j, s):
        return s + jnp.sum(jnp.exp(out_ref[pl.ds(_NBTC + j, 1)] - m))

    s2 = lax.fori_loop(0, _NBSC, _asum, jnp.float32(0.0))
    scale1 = jnp.exp(jnp.broadcast_to(m1 - m, (1, 128)))[0, 0]
    s = s1 * scale1 + s2
    lse = m + jnp.log(jnp.broadcast_to(s, (1, 128)))[0, 0]

    def _fin_tc(j, carry):
        out_ref[pl.ds(j, 1)] = ltc_ref[pl.ds(j, 1)] - lse
        return carry

    lax.fori_loop(0, _NBTC, _fin_tc, 0)

    def _fin_sc(j, carry):
        out_ref[pl.ds(_NBTC + j, 1)] = out_ref[pl.ds(_NBTC + j, 1)] - lse
        return carry

    lax.fori_loop(0, _NBSC, _fin_sc, 0)


def _combine(ltc, lsc3, b2r, st):
    return pl.pallas_call(
        _combine_body,
        out_shape=jax.ShapeDtypeStruct((_NB, 1, _BLK), jnp.float32),
    )(ltc, lsc3, b2r, st)


def kernel(inp, emb, W1, b1, W2, b2):
    gathered = _sc_gather(inp.astype(jnp.int32), emb)
    x = gathered.reshape(1, _CTX2 * _EMB)
    h = _tc_h(x, W1, b1.reshape(1, _HID))
    b2r = b2.reshape(_NB, 1, _BLK)
    lsc = _sc_logits(h.reshape(_HID), W2)
    ltc, st = _tc_logits(h.astype(jnp.bfloat16), W2, b2r)
    out = _combine(ltc, lsc.reshape(_NBSC, 1, _BLK), b2r, st)
    return out.reshape(1, _VOCAB)


# speedup vs baseline: 1.2812x; 1.2812x over previous
import jax
import jax.numpy as jnp
from jax import lax
from jax.experimental import pallas as pl
from jax.experimental.pallas import tpu as pltpu
from jax.experimental.pallas import tpu_sc as plsc

_VOCAB = 100000
_EMB = 64
_CTX2 = 20
_HID = 512

_BLK = 4000
_VTC = 80000
_VSC = 20480
_SC_START = _VOCAB - _VSC
_SC_SKIP = _VTC - _SC_START
_NBTC = _VTC // _BLK
_NBSC = (_VOCAB - _VTC) // _BLK
_NB = _VOCAB // _BLK

_NTEC = 32
_NR = _VSC // _NTEC
_CH = 40
_NCH = _NR // _CH
assert _NR % _CH == 0 and _CH % 8 == 0 and _SC_START % 8 == 0


def _gather_body(idx_hbm, table_hbm, out_hbm, idx_v, rows_v, sem):
    wid = lax.axis_index("s") * 2 + lax.axis_index("c")

    @pl.when(wid == 0)
    def _():
        pltpu.sync_copy(idx_hbm, idx_v)
        v_lo = idx_v[pl.ds(0, 16)]
        v_hi = idx_v[pl.ds(4, 16)]
        copies = []
        for j in range(_CTX2):
            r = v_lo[j] if j < 16 else v_hi[j - 4]
            copies.append(pltpu.async_copy(
                table_hbm.at[pl.ds(r, 1)], rows_v.at[pl.ds(j, 1)], sem))
        for c in copies:
            c.wait()
        pltpu.sync_copy(rows_v, out_hbm)


def _sc_gather(idx, emb):
    mesh = plsc.VectorSubcoreMesh(core_axis_name="c", subcore_axis_name="s")
    return pl.kernel(
        _gather_body,
        out_type=jax.ShapeDtypeStruct((_CTX2, _EMB), jnp.float32),
        mesh=mesh,
        scratch_types=[
            pltpu.VMEM((_CTX2,), jnp.int32),
            pltpu.VMEM((_CTX2, _EMB), jnp.float32),
            pltpu.SemaphoreType.DMA,
        ],
    )(idx, emb)


def _h_body(x_ref, w1_ref, b1_ref, h_ref):
    v = lax.dot_general(x_ref[...], w1_ref[...], (((1,), (1,)), ((), ())),
                        preferred_element_type=jnp.float32)
    h_ref[...] = jnp.maximum(v + b1_ref[...], 0.0)


def _tc_h(x, W1, b1):
    return pl.pallas_call(
        _h_body,
        out_shape=jax.ShapeDtypeStruct((1, _HID), jnp.float32),
    )(x, W1, b1)


def _sc_logits_body(h_hbm, w2_hbm, out_hbm, h_v, buf, lg, tp, sem0, sem1):
    wid = lax.axis_index("s") * 2 + lax.axis_index("c")
    base = _SC_START + wid * _NR
    obase = wid * _NR
    pltpu.sync_copy(h_hbm, h_v)
    hv = [h_v[pl.ds(c * 16, 16)] for c in range(_HID // 16)]
    iota = lax.iota(jnp.int32, 16)

    group_starts = []
    g = 0
    while g + 16 <= _CH:
        group_starts.append(g)
        g += 16
    if _CH % 16:
        group_starts.append(_CH - 16)

    def _start(ci, p):
        sem = [sem0, sem1][p]
        pltpu.async_copy(
            w2_hbm.at[pl.ds(base + ci * _CH, _CH)], buf.at[p], sem)

    def _wait(p):
        sem = [sem0, sem1][p]
        pltpu.make_async_copy(
            w2_hbm.at[pl.ds(base, _CH)], buf.at[p], sem).wait()

    _start(0, 0)
    _start(1, 1)

    def _chunk(ci, p):
        _wait(p)
        for gs in group_starts:
            accs = []
            for r in range(16):
                acc = jnp.zeros((16,), jnp.float32)
                for c in range(_HID // 16):
                    w = buf[p, gs + r, pl.ds(c * 16, 16)]
                    acc = acc + w * hv[c]
                accs.append(acc)
            for r in range(16):
                plsc.store_scatter(
                    tp, [iota, jnp.full((16,), r, jnp.int32)], accs[r])
            vsum = tp[0]
            for j in range(1, 16):
                vsum = vsum + tp[j]
            lg[pl.ds(gs, 16)] = vsum

        @pl.when(ci + 2 < _NCH)
        def _():
            _start(ci + 2, p)

        pltpu.sync_copy(
            lg, out_hbm.at[pl.ds(obase + ci * _CH, _CH)])

    def _loop_body(ci, carry):
        @pl.when(ci % 2 == 0)
        def _():
            _chunk(ci, 0)

        @pl.when(ci % 2 == 1)
        def _():
            _chunk(ci, 1)

        return carry

    lax.fori_loop(0, _NCH, _loop_body, 0)


def _sc_logits(h, W2):
    mesh = plsc.VectorSubcoreMesh(core_axis_name="c", subcore_axis_name="s")
    return pl.kernel(
        _sc_logits_body,
        out_type=jax.ShapeDtypeStruct((_VSC,), jnp.float32),
        mesh=mesh,
        compiler_params=pltpu.CompilerParams(needs_layout_passes=False),
        scratch_types=[
            pltpu.VMEM((_HID,), jnp.float32),
            pltpu.VMEM((2, _CH, _HID), jnp.float32),
            pltpu.VMEM((_CH,), jnp.float32),
            pltpu.VMEM((16, 16), jnp.float32),
            pltpu.SemaphoreType.DMA,
            pltpu.SemaphoreType.DMA,
        ],
    )(h, W2)


def _mlp_body(h_ref, w2_ref, b2_ref, out_ref, st_ref, ms_ref):
    i = pl.program_id(0)

    @pl.when(i == 0)
    def _():
        ms_ref[0] = jnp.float32(-jnp.inf)
        ms_ref[1] = jnp.float32(0.0)

    logits = lax.dot_general(
        h_ref[...], w2_ref[...].astype(jnp.bfloat16), (((1,), (1,)), ((), ())),
        preferred_element_type=jnp.float32)
    blk = logits + b2_ref[0]
    out_ref[pl.ds(i, 1)] = blk.reshape(1, 1, _BLK)
    m_old = ms_ref[0]
    m_new = jnp.maximum(m_old, jnp.max(blk))
    blk_sum = jnp.sum(jnp.exp(blk - m_new))
    scale = jnp.exp(jnp.broadcast_to(m_old - m_new, (1, 128)))[0, 0]
    ms_ref[0] = m_new
    ms_ref[1] = ms_ref[1] * scale + blk_sum

    @pl.when(i == _NBTC - 1)
    def _():
        ii = lax.broadcasted_iota(jnp.int32, (1, 128), 1)
        st_ref[...] = jnp.where(ii == 0, ms_ref[0], ms_ref[1])


def _tc_logits(h_bf, W2, b2r):
    return pl.pallas_call(
        _mlp_body,
        grid=(_NBTC,),
        in_specs=[
            pl.BlockSpec((1, _HID), lambda i: (0, 0)),
            pl.BlockSpec((_BLK, _HID), lambda i: (i, 0)),
            pl.BlockSpec((1, 1, _BLK), lambda i: (i, 0, 0)),
        ],
        out_specs=[
            pl.BlockSpec((_NBTC, 1, _BLK), lambda i: (0, 0, 0)),
            pl.BlockSpec((1, 128), lambda i: (0, 0)),
        ],
        out_shape=[
            jax.ShapeDtypeStruct((_NBTC, 1, _BLK), jnp.float32),
            jax.ShapeDtypeStruct((1, 128), jnp.float32),
        ],
        scratch_shapes=[pltpu.SMEM((2,), jnp.float32)],
    )(h_bf, W2, b2r)


def _combine_body(ltc_ref, lsc_ref, b2_ref, st_ref, out_ref):
    m1 = st_ref[0, 0]
    s1 = st_ref[0, 1]

    def _amax(j, m2):
        c = lsc_ref[pl.ds(j, 1)] + b2_ref[pl.ds(_NBTC + j, 1)]
        out_ref[pl.ds(_NBTC + j, 1)] = c
        return jnp.maximum(m2, jnp.max(c))

    m2 = lax.fori_loop(0, _NBSC, _amax, jnp.float32(-jnp.inf))
    m = jnp.maximum(m1, m2)

    def _asum(j, s):
        return s + jnp.sum(jnp.exp(out_ref[pl.ds(_NBTC + j, 1)] - m))

    s2 = lax.fori_loop(0, _NBSC, _asum, jnp.float32(0.0))
    scale1 = jnp.exp(jnp.broadcast_to(m1 - m, (1, 128)))[0, 0]
    s = s1 * scale1 + s2
    lse = m + jnp.log(jnp.broadcast_to(s, (1, 128)))[0, 0]

    def _fin_tc(j, carry):
        out_ref[pl.ds(j, 1)] = ltc_ref[pl.ds(j, 1)] - lse
        return carry

    lax.fori_loop(0, _NBTC, _fin_tc, 0)

    def _fin_sc(j, carry):
        out_ref[pl.ds(_NBTC + j, 1)] = out_ref[pl.ds(_NBTC + j, 1)] - lse
        return carry

    lax.fori_loop(0, _NBSC, _fin_sc, 0)


def _combine(ltc, lsc3, b2r, st):
    return pl.pallas_call(
        _combine_body,
        out_shape=jax.ShapeDtypeStruct((_NB, 1, _BLK), jnp.float32),
    )(ltc, lsc3, b2r, st)


def kernel(inp, emb, W1, b1, W2, b2):
    gathered = _sc_gather(inp.astype(jnp.int32), emb)
    x = gathered.reshape(1, _CTX2 * _EMB)
    h = _tc_h(x, W1, b1.reshape(1, _HID))
    b2r = b2.reshape(_NB, 1, _BLK)
    lsc = _sc_logits(h.reshape(_HID), W2)
    ltc, st = _tc_logits(h.astype(jnp.bfloat16), W2, b2r)
    lsc3 = lsc[_SC_SKIP:].reshape(_NBSC, 1, _BLK)
    out = _combine(ltc, lsc3, b2r, st)
    return out.reshape(1, _VOCAB)


# speedup vs baseline: 1.5712x; 1.2263x over previous
import jax
import jax.numpy as jnp
from jax import lax
from jax.experimental import pallas as pl
from jax.experimental.pallas import tpu as pltpu
from jax.experimental.pallas import tpu_sc as plsc

_VOCAB = 100000
_EMB = 64
_CTX2 = 20
_HID = 512

_BLK = 4000
_VTC = 84000
_VSC = 16384
_SC_START = _VOCAB - _VSC
_SC_SKIP = _VTC - _SC_START
_NBTC = _VTC // _BLK
_NBSC = (_VOCAB - _VTC) // _BLK
_NB = _VOCAB // _BLK

_NTEC = 32
_NR = _VSC // _NTEC
_CH = 32
_NCH = _NR // _CH
assert _NR % _CH == 0 and _CH % 8 == 0 and _SC_START % 8 == 0


def _gather_body(idx_ref, emb_hbm, out_ref, rows_ref, sem):
    copies = [
        pltpu.make_async_copy(
            emb_hbm.at[pl.ds(idx_ref[j], 1), :],
            rows_ref.at[pl.ds(j, 1), :], sem)
        for j in range(_CTX2)
    ]
    for c in copies:
        c.start()
    for c in copies:
        c.wait()
    out_ref[...] = rows_ref[...]


def _tc_gather(idx, emb):
    return pl.pallas_call(
        _gather_body,
        in_specs=[
            pl.BlockSpec(memory_space=pltpu.SMEM),
            pl.BlockSpec(memory_space=pltpu.HBM),
        ],
        out_shape=jax.ShapeDtypeStruct((_CTX2, _EMB), jnp.float32),
        scratch_shapes=[pltpu.VMEM((_CTX2, _EMB), jnp.float32),
                        pltpu.SemaphoreType.DMA],
    )(idx, emb)


def _h_body(x_ref, w1_ref, b1_ref, h_ref):
    v = lax.dot_general(x_ref[...], w1_ref[...], (((1,), (1,)), ((), ())),
                        preferred_element_type=jnp.float32)
    h_ref[...] = jnp.maximum(v + b1_ref[...], 0.0)


def _tc_h(x, W1, b1):
    return pl.pallas_call(
        _h_body,
        out_shape=jax.ShapeDtypeStruct((1, _HID), jnp.float32),
    )(x, W1, b1)


def _sc_logits_body(h_hbm, w2_hbm, out_hbm, h_v, buf, lg, tp, sem0, sem1):
    wid = lax.axis_index("s") * 2 + lax.axis_index("c")
    base = _SC_START + wid * _NR
    obase = wid * _NR
    pltpu.sync_copy(h_hbm, h_v)
    hv = [h_v[pl.ds(c * 16, 16)] for c in range(_HID // 16)]
    iota = lax.iota(jnp.int32, 16)

    group_starts = []
    g = 0
    while g + 16 <= _CH:
        group_starts.append(g)
        g += 16
    if _CH % 16:
        group_starts.append(_CH - 16)

    def _start(ci, p):
        sem = [sem0, sem1][p]
        pltpu.async_copy(
            w2_hbm.at[pl.ds(base + ci * _CH, _CH)], buf.at[p], sem)

    def _wait(p):
        sem = [sem0, sem1][p]
        pltpu.make_async_copy(
            w2_hbm.at[pl.ds(base, _CH)], buf.at[p], sem).wait()

    _start(0, 0)
    _start(1, 1)

    def _chunk(ci, p):
        _wait(p)
        for gs in group_starts:
            accs = []
            for r in range(16):
                acc = jnp.zeros((16,), jnp.float32)
                for c in range(_HID // 16):
                    w = buf[p, gs + r, pl.ds(c * 16, 16)]
                    acc = acc + w * hv[c]
                accs.append(acc)
            for r in range(16):
                plsc.store_scatter(
                    tp, [iota, jnp.full((16,), r, jnp.int32)], accs[r])
            vsum = tp[0]
            for j in range(1, 16):
                vsum = vsum + tp[j]
            lg[pl.ds(gs, 16)] = vsum

        @pl.when(ci + 2 < _NCH)
        def _():
            _start(ci + 2, p)

        pltpu.sync_copy(
            lg, out_hbm.at[pl.ds(obase + ci * _CH, _CH)])

    def _loop_body(ci, carry):
        @pl.when(ci % 2 == 0)
        def _():
            _chunk(ci, 0)

        @pl.when(ci % 2 == 1)
        def _():
            _chunk(ci, 1)

        return carry

    lax.fori_loop(0, _NCH, _loop_body, 0)


def _sc_logits(h, W2):
    mesh = plsc.VectorSubcoreMesh(core_axis_name="c", subcore_axis_name="s")
    return pl.kernel(
        _sc_logits_body,
        out_type=jax.ShapeDtypeStruct((_VSC,), jnp.float32),
        mesh=mesh,
        compiler_params=pltpu.CompilerParams(needs_layout_passes=False),
        scratch_types=[
            pltpu.VMEM((_HID,), jnp.float32),
            pltpu.VMEM((2, _CH, _HID), jnp.float32),
            pltpu.VMEM((_CH,), jnp.float32),
            pltpu.VMEM((16, 16), jnp.float32),
            pltpu.SemaphoreType.DMA,
            pltpu.SemaphoreType.DMA,
        ],
    )(h, W2)


def _mlp_body(h_ref, w2_ref, b2_ref, out_ref, st_ref, ms_ref):
    i = pl.program_id(0)

    @pl.when(i == 0)
    def _():
        ms_ref[0] = jnp.float32(-jnp.inf)
        ms_ref[1] = jnp.float32(0.0)

    logits = lax.dot_general(
        h_ref[...], w2_ref[...].astype(jnp.bfloat16), (((1,), (1,)), ((), ())),
        preferred_element_type=jnp.float32)
    blk = logits + b2_ref[0]
    out_ref[pl.ds(i, 1)] = blk.reshape(1, 1, _BLK)
    m_old = ms_ref[0]
    m_new = jnp.maximum(m_old, jnp.max(blk))
    blk_sum = jnp.sum(jnp.exp(blk - m_new))
    scale = jnp.exp(jnp.broadcast_to(m_old - m_new, (1, 128)))[0, 0]
    ms_ref[0] = m_new
    ms_ref[1] = ms_ref[1] * scale + blk_sum

    @pl.when(i == _NBTC - 1)
    def _():
        ii = lax.broadcasted_iota(jnp.int32, (1, 128), 1)
        st_ref[...] = jnp.where(ii == 0, ms_ref[0], ms_ref[1])


def _tc_logits(h_bf, W2, b2r):
    return pl.pallas_call(
        _mlp_body,
        grid=(_NBTC,),
        in_specs=[
            pl.BlockSpec((1, _HID), lambda i: (0, 0)),
            pl.BlockSpec((_BLK, _HID), lambda i: (i, 0)),
            pl.BlockSpec((1, 1, _BLK), lambda i: (i, 0, 0)),
        ],
        out_specs=[
            pl.BlockSpec((_NBTC, 1, _BLK), lambda i: (0, 0, 0)),
            pl.BlockSpec((1, 128), lambda i: (0, 0)),
        ],
        out_shape=[
            jax.ShapeDtypeStruct((_NBTC, 1, _BLK), jnp.float32),
            jax.ShapeDtypeStruct((1, 128), jnp.float32),
        ],
        scratch_shapes=[pltpu.SMEM((2,), jnp.float32)],
    )(h_bf, W2, b2r)


def _combine_body(ltc_ref, lsc_ref, b2_ref, st_ref, out_ref):
    m1 = st_ref[0, 0]
    s1 = st_ref[0, 1]

    def _amax(j, m2):
        c = lsc_ref[pl.ds(j, 1)] + b2_ref[pl.ds(_NBTC + j, 1)]
        out_ref[pl.ds(_NBTC + j, 1)] = c
        return jnp.maximum(m2, jnp.max(c))

    m2 = lax.fori_loop(0, _NBSC, _amax, jnp.float32(-jnp.inf))
    m = jnp.maximum(m1, m2)

    def _asum(j, s):
        return s + jnp.sum(jnp.exp(out_ref[pl.ds(_NBTC + j, 1)] - m))

    s2 = lax.fori_loop(0, _NBSC, _asum, jnp.float32(0.0))
    scale1 = jnp.exp(jnp.broadcast_to(m1 - m, (1, 128)))[0, 0]
    s = s1 * scale1 + s2
    lse = m + jnp.log(jnp.broadcast_to(s, (1, 128)))[0, 0]

    def _fin_tc(j, carry):
        out_ref[pl.ds(j, 1)] = ltc_ref[pl.ds(j, 1)] - lse
        return carry

    lax.fori_loop(0, _NBTC, _fin_tc, 0)

    def _fin_sc(j, carry):
        out_ref[pl.ds(_NBTC + j, 1)] = out_ref[pl.ds(_NBTC + j, 1)] - lse
        return carry

    lax.fori_loop(0, _NBSC, _fin_sc, 0)


def _combine(ltc, lsc3, b2r, st):
    return pl.pallas_call(
        _combine_body,
        out_shape=jax.ShapeDtypeStruct((_NB, 1, _BLK), jnp.float32),
    )(ltc, lsc3, b2r, st)


def kernel(inp, emb, W1, b1, W2, b2):
    gathered = _tc_gather(inp.astype(jnp.int32), emb)
    x = gathered.reshape(1, _CTX2 * _EMB)
    h = _tc_h(x, W1, b1.reshape(1, _HID))
    b2r = b2.reshape(_NB, 1, _BLK)
    lsc = _sc_logits(h.reshape(_HID), W2)
    ltc, st = _tc_logits(h.astype(jnp.bfloat16), W2, b2r)
    lsc3 = lsc[_SC_SKIP:].reshape(_NBSC, 1, _BLK)
    out = _combine(ltc, lsc3, b2r, st)
    return out.reshape(1, _VOCAB)


# speedup vs baseline: 1.8600x; 1.1838x over previous
import jax
import jax.numpy as jnp
from jax import lax
from jax.experimental import pallas as pl
from jax.experimental.pallas import tpu as pltpu

_VOCAB = 100000
_EMB = 64
_CTX2 = 20
_HID = 512

_BLK = 4000
_NB = _VOCAB // _BLK


def _gather_body(idx_ref, emb_hbm, out_ref, sem):
    copies = [
        pltpu.make_async_copy(
            emb_hbm.at[pl.ds(idx_ref[j], 1), :],
            out_ref.at[pl.ds(j, 1), :], sem)
        for j in range(_CTX2)
    ]
    for c in copies:
        c.start()
    for c in copies:
        c.wait()


def _tc_gather(idx, emb):
    return pl.pallas_call(
        _gather_body,
        in_specs=[
            pl.BlockSpec(memory_space=pltpu.SMEM),
            pl.BlockSpec(memory_space=pltpu.HBM),
        ],
        out_shape=jax.ShapeDtypeStruct((_CTX2, _EMB), jnp.float32),
        scratch_shapes=[pltpu.SemaphoreType.DMA],
    )(idx, emb)


def _mlp_body(x_ref, w1_ref, b1_ref, w2_ref, b2_ref, out_ref, h_ref, ms_ref):
    i = pl.program_id(0)

    @pl.when(i == 0)
    def _():
        h = lax.dot_general(
            x_ref[...], w1_ref[...], (((1,), (1,)), ((), ())),
            preferred_element_type=jnp.float32)
        h_ref[...] = jnp.maximum(h + b1_ref[...], 0.0).astype(jnp.bfloat16)
        ms_ref[0] = jnp.float32(-jnp.inf)
        ms_ref[1] = jnp.float32(0.0)

    logits = lax.dot_general(
        h_ref[...], w2_ref[...].astype(jnp.bfloat16), (((1,), (1,)), ((), ())),
        preferred_element_type=jnp.float32)
    blk = logits + b2_ref[0]
    out_ref[pl.ds(i, 1)] = blk.reshape(1, 1, _BLK)
    m_old = ms_ref[0]
    m_new = jnp.maximum(m_old, jnp.max(blk))
    blk_sum = jnp.sum(jnp.exp(blk - m_new))
    scale = jnp.exp(jnp.broadcast_to(m_old - m_new, (1, 128)))[0, 0]
    ms_ref[0] = m_new
    ms_ref[1] = ms_ref[1] * scale + blk_sum

    @pl.when(i == _NB - 1)
    def _():
        m = ms_ref[0]
        s = ms_ref[1]

        def _sub_body(j, carry):
            c = out_ref[pl.ds(j, 1)]
            out_ref[pl.ds(j, 1)] = c - m - jnp.log(jnp.broadcast_to(s, c.shape))
            return carry

        lax.fori_loop(0, _NB, _sub_body, 0)


def _mlp_logsoftmax(x, W1, b1, W2, b2):
    return pl.pallas_call(
        _mlp_body,
        grid=(_NB,),
        in_specs=[
            pl.BlockSpec((1, _CTX2 * _EMB), lambda i: (0, 0)),
            pl.BlockSpec((_HID, _CTX2 * _EMB), lambda i: (0, 0)),
            pl.BlockSpec((1, _HID), lambda i: (0, 0)),
            pl.BlockSpec((_BLK, _HID), lambda i: (i, 0)),
            pl.BlockSpec((1, 1, _BLK), lambda i: (i, 0, 0)),
        ],
        out_specs=pl.BlockSpec((_NB, 1, _BLK), lambda i: (0, 0, 0)),
        out_shape=jax.ShapeDtypeStruct((_NB, 1, _BLK), jnp.float32),
        scratch_shapes=[pltpu.VMEM((1, _HID), jnp.bfloat16),
                        pltpu.SMEM((2,), jnp.float32)],
    )(x, W1, b1, W2, b2)


def kernel(inp, emb, W1, b1, W2, b2):
    gathered = _tc_gather(inp.astype(jnp.int32), emb)
    x = gathered.reshape(1, _CTX2 * _EMB)
    out = _mlp_logsoftmax(x, W1, b1.reshape(1, _HID), W2,
                          b2.reshape(_NB, 1, _BLK))
    return out.reshape(1, _VOCAB)
